# baseline (speedup 1.0000x reference)
import jax
import jax.numpy as jnp
from jax import lax
from jax.experimental import pallas as pl
from jax.experimental.pallas import tpu as pltpu
from jax.experimental.pallas import tpu_sc as plsc

NC = 2
NS = 16
NW = NC * NS
CH = 128
NBUF = 4


def _make_gather(n_ids: int, d: int, interpret: bool = False):
    assert n_ids % (NW * CH) == 0
    cpw = n_ids // (NW * CH)
    assert cpw % NBUF == 0
    b_per_w = cpw * CH
    mesh = plsc.VectorSubcoreMesh(
        core_axis_name="c", subcore_axis_name="s", num_cores=NC, num_subcores=NS
    )

    def body(idx_hbm, table_hbm, out_hbm, idx_v, rows_v, *sems):
        gsem = sems[:NBUF]
        osem = sems[NBUF:]
        wid = lax.axis_index("s") * NC + lax.axis_index("c")
        wbase = wid * b_per_w
        pltpu.sync_copy(idx_hbm.at[wid], idx_v)

        for b in range(NBUF):
            pltpu.async_copy(table_hbm.at[idx_v.at[b]], rows_v.at[b], gsem[b])

        @pl.loop(0, cpw // NBUF)
        def _(g):
            for b in range(NBUF):
                j = g * NBUF + b
                pltpu.make_async_copy(
                    table_hbm.at[idx_v.at[0]], rows_v.at[b], gsem[b]
                ).wait()
                pltpu.async_copy(
                    rows_v.at[b], out_hbm.at[pl.ds(wbase + j * CH, CH)], osem[b]
                )
                pltpu.make_async_copy(
                    table_hbm.at[idx_v.at[0]], rows_v.at[b], osem[b]
                ).wait()

                @pl.when(j + NBUF < cpw)
                def _():
                    pltpu.async_copy(
                        table_hbm.at[idx_v.at[j + NBUF]], rows_v.at[b], gsem[b]
                    )

    return pl.kernel(
        body,
        out_type=jax.ShapeDtypeStruct((n_ids, d), jnp.float32),
        mesh=mesh,
        scratch_types=(
            [
                pltpu.VMEM((cpw, CH), jnp.int32),
                pltpu.VMEM((NBUF, CH, d), jnp.float32),
            ]
            + [pltpu.SemaphoreType.DMA] * (2 * NBUF)
        ),
        compiler_params=pltpu.CompilerParams(use_tc_tiling_on_sc=False),
        interpret=interpret,
    )


def kernel(token_ids, embedding_matrix):
    b, t = token_ids.shape
    n = b * t
    d = embedding_matrix.shape[1]
    idx = token_ids.astype(jnp.int32).reshape(NW, n // (NW * CH), CH)
    out = _make_gather(n, d)(idx, embedding_matrix)
    return out.reshape(b, t, d)
